# Initial kernel scaffold; baseline (speedup 1.0000x reference)
import jax
import jax.numpy as jnp
from jax.experimental import pallas as pl


def kernel(x, Wg, W1, b1, W2, b2):
    raise NotImplementedError("write your pallas kernel here")


# speedup vs baseline: 1.6207x; 1.6207x over previous
import functools

import jax
import jax.numpy as jnp
from jax import lax
from jax.experimental import pallas as pl
from jax.experimental.pallas import tpu as pltpu
from jax.experimental.pallas import tpu_sc as plsc

N_TOK = 32768
D_MODEL = 768
NUM_EXPERTS = 8
HID = 384
TOP_K = 2

BLK = 256
NTILES = 2 * N_TOK // BLK + NUM_EXPERTS
MAXROWS = NTILES * BLK
GATE_TILE = 256
NGATE = N_TOK // GATE_TILE

NC, NS = 2, 16
NW = NC * NS
TPW = N_TOK // NW
SUB = 32
NSUB = TPW // SUB

_SQRT1_2 = 0.7071067811865476


def _gate_kernel(x_ref, wg_ref, e1_ref, e2_ref, r0_ref, r1_ref, ws_ref,
                 poff_ref, te_ref, counts):
    i = pl.program_id(0)

    @pl.when(i == 0)
    def _():
        counts[...] = jnp.zeros_like(counts)

    xb = x_ref[...]
    wg = wg_ref[...]
    logits = lax.dot_general(xb, wg, (((1,), (0,)), ((), ())),
                             precision=lax.Precision.DEFAULT)
    m = jnp.max(logits, axis=-1, keepdims=True)
    Z = jnp.sum(jnp.exp(logits - m), axis=-1, keepdims=True)
    iota_e = lax.broadcasted_iota(jnp.int32, (GATE_TILE, NUM_EXPERTS), 1)
    v1 = jnp.max(logits, axis=-1, keepdims=True)
    e1 = jnp.min(jnp.where(logits == v1, iota_e, NUM_EXPERTS), axis=-1)
    l2 = jnp.where(iota_e == e1[:, None], -jnp.inf, logits)
    v2 = jnp.max(l2, axis=-1, keepdims=True)
    e2 = jnp.min(jnp.where(l2 == v2, iota_e, NUM_EXPERTS), axis=-1)
    ws = (jnp.exp(v1 - m) + jnp.exp(v2 - m)) / Z

    iota_w = lax.broadcasted_iota(jnp.int32, (GATE_TILE, 128), 1)
    oh1 = (iota_w == e1[:, None]).astype(jnp.float32)
    oh2 = (iota_w == e2[:, None]).astype(jnp.float32)
    asg = oh1 + oh2
    ri = lax.broadcasted_iota(jnp.int32, (GATE_TILE, GATE_TILE), 0)
    ci = lax.broadcasted_iota(jnp.int32, (GATE_TILE, GATE_TILE), 1)
    tril = (ci <= ri).astype(jnp.float32)
    cin = lax.dot_general(tril, asg, (((1,), (0,)), ((), ())),
                          precision=lax.Precision.HIGHEST)
    cex = cin - asg
    carry = counts[...].astype(jnp.float32)
    r0 = jnp.sum(oh1 * (cex + carry), axis=-1)
    r1 = jnp.sum(oh2 * (cex + carry), axis=-1)
    counts[...] = counts[...] + jnp.sum(asg, axis=0, keepdims=True).astype(jnp.int32)

    e1_ref[...] = e1[None, None, :]
    e2_ref[...] = e2[None, None, :]
    r0_ref[...] = r0.astype(jnp.int32)[None, None, :]
    r1_ref[...] = r1.astype(jnp.int32)[None, None, :]
    ws_ref[...] = ws[:, 0][None, None, :]

    @pl.when(i == NGATE - 1)
    def _():
        cnt = counts[...]
        padded = ((cnt + (BLK - 1)) // BLK) * BLK
        fi = lax.broadcasted_iota(jnp.int32, (128, 128), 0)
        ei = lax.broadcasted_iota(jnp.int32, (128, 128), 1)
        strict = (fi < ei).astype(jnp.float32)
        poff = lax.dot_general(padded.astype(jnp.float32), strict,
                               (((1,), (0,)), ((), ())),
                               precision=lax.Precision.HIGHEST)
        poffi = poff.astype(jnp.int32)
        poff_ref[...] = poffi
        bt = poffi // BLK
        lane = lax.broadcasted_iota(jnp.int32, (1, 128), 1)
        i512 = lax.broadcasted_iota(jnp.int32, (1, 512), 1)
        s = jnp.zeros((1, 512), jnp.int32)
        for e in range(NUM_EXPERTS):
            be = jnp.sum(jnp.where(lane == e, bt, 0))
            s = s + (i512 >= be).astype(jnp.int32)
        te_ref[...] = jnp.clip(s - 1, 0, NUM_EXPERTS - 1)


def _gating(x, Wg):
    shp = jax.ShapeDtypeStruct
    outs = pl.pallas_call(
        _gate_kernel,
        grid=(NGATE,),
        in_specs=[
            pl.BlockSpec((GATE_TILE, D_MODEL), lambda i: (i, 0)),
            pl.BlockSpec((D_MODEL, NUM_EXPERTS), lambda i: (0, 0)),
        ],
        out_specs=[
            pl.BlockSpec((1, 1, GATE_TILE), lambda i: (i, 0, 0)),
            pl.BlockSpec((1, 1, GATE_TILE), lambda i: (i, 0, 0)),
            pl.BlockSpec((1, 1, GATE_TILE), lambda i: (i, 0, 0)),
            pl.BlockSpec((1, 1, GATE_TILE), lambda i: (i, 0, 0)),
            pl.BlockSpec((1, 1, GATE_TILE), lambda i: (i, 0, 0)),
            pl.BlockSpec((1, 128), lambda i: (0, 0)),
            pl.BlockSpec((1, 512), lambda i: (0, 0)),
        ],
        out_shape=[
            shp((NGATE, 1, GATE_TILE), jnp.int32),
            shp((NGATE, 1, GATE_TILE), jnp.int32),
            shp((NGATE, 1, GATE_TILE), jnp.int32),
            shp((NGATE, 1, GATE_TILE), jnp.int32),
            shp((NGATE, 1, GATE_TILE), jnp.float32),
            shp((1, 128), jnp.int32),
            shp((1, 512), jnp.int32),
        ],
        scratch_shapes=[pltpu.VMEM((1, 128), jnp.int32)],
    )(x, Wg)
    return outs


def _disp_body(x_hbm, e1_hbm, e2_hbm, r0_hbm, r1_hbm, poff_hbm,
               xs_hbm, d0_hbm, d1_hbm,
               poff_v, e1_v, e2_v, r0_v, r1_v, idx0_sub, idx1_sub, rows_v,
               sem0, sem1):
    wid = lax.axis_index("s") * NC + lax.axis_index("c")
    base = wid * TPW
    pltpu.sync_copy(poff_hbm, poff_v)
    pltpu.sync_copy(e1_hbm.at[pl.ds(base, TPW)], e1_v)
    pltpu.sync_copy(e2_hbm.at[pl.ds(base, TPW)], e2_v)
    pltpu.sync_copy(r0_hbm.at[pl.ds(base, TPW)], r0_v)
    pltpu.sync_copy(r1_hbm.at[pl.ds(base, TPW)], r1_v)

    def sub(j, carry):
        for k in range(SUB // 16):
            sl = pl.ds(j * SUB + k * 16, 16)
            ko = pl.ds(k * 16, 16)
            idx0_sub[ko] = plsc.load_gather(poff_v, [e1_v[sl]]) + r0_v[sl]
            idx1_sub[ko] = plsc.load_gather(poff_v, [e2_v[sl]]) + r1_v[sl]
        pltpu.sync_copy(idx0_sub, d0_hbm.at[wid, j])
        pltpu.sync_copy(idx1_sub, d1_hbm.at[wid, j])
        pltpu.sync_copy(x_hbm.at[pl.ds(base + j * SUB, SUB)], rows_v)
        cp0 = pltpu.async_copy(rows_v, xs_hbm.at[idx0_sub], sem0)
        cp1 = pltpu.async_copy(rows_v, xs_hbm.at[idx1_sub], sem1)
        cp0.wait()
        cp1.wait()
        return carry

    lax.fori_loop(0, NSUB, sub, 0)


def _dispatch(x, e1, e2, r0, r1, poff):
    shp = jax.ShapeDtypeStruct
    mesh = plsc.VectorSubcoreMesh(core_axis_name="c", subcore_axis_name="s")
    f = pl.kernel(
        _disp_body,
        out_type=(
            shp((MAXROWS, D_MODEL), jnp.float32),
            shp((NW, NSUB, SUB), jnp.int32),
            shp((NW, NSUB, SUB), jnp.int32),
        ),
        mesh=mesh,
        scratch_types=[
            pltpu.VMEM((128,), jnp.int32),
            pltpu.VMEM((TPW,), jnp.int32),
            pltpu.VMEM((TPW,), jnp.int32),
            pltpu.VMEM((TPW,), jnp.int32),
            pltpu.VMEM((TPW,), jnp.int32),
            pltpu.VMEM((SUB,), jnp.int32),
            pltpu.VMEM((SUB,), jnp.int32),
            pltpu.VMEM((SUB, D_MODEL), jnp.float32),
            pltpu.SemaphoreType.DMA,
            pltpu.SemaphoreType.DMA,
        ],
        compiler_params=pltpu.CompilerParams(needs_layout_passes=False),
    )
    return f(x, e1, e2, r0, r1, poff)


def _ffn_kernel(te_ref, xs_ref, w1_ref, b1_ref, w2_ref, b2_ref, o_ref):
    xb = xs_ref[...].astype(jnp.bfloat16)
    w1 = w1_ref[0].astype(jnp.bfloat16)
    h = lax.dot_general(xb, w1, (((1,), (0,)), ((), ())),
                        preferred_element_type=jnp.float32)
    h = h + b1_ref[0]
    h = 0.5 * h * (1.0 + lax.erf(h * _SQRT1_2))
    hb = h.astype(jnp.bfloat16)
    w2 = w2_ref[0].astype(jnp.bfloat16)
    o = lax.dot_general(hb, w2, (((1,), (0,)), ((), ())),
                        preferred_element_type=jnp.float32)
    o_ref[...] = o + b2_ref[0]


def _grouped_ffn(te, xs, W1, b1, W2, b2):
    grid_spec = pltpu.PrefetchScalarGridSpec(
        num_scalar_prefetch=1,
        grid=(NTILES,),
        in_specs=[
            pl.BlockSpec((BLK, D_MODEL), lambda i, te: (i, 0)),
            pl.BlockSpec((1, D_MODEL, HID), lambda i, te: (te[i], 0, 0)),
            pl.BlockSpec((1, 1, HID), lambda i, te: (te[i], 0, 0)),
            pl.BlockSpec((1, HID, D_MODEL), lambda i, te: (te[i], 0, 0)),
            pl.BlockSpec((1, 1, D_MODEL), lambda i, te: (te[i], 0, 0)),
        ],
        out_specs=pl.BlockSpec((BLK, D_MODEL), lambda i, te: (i, 0)),
    )
    return pl.pallas_call(
        _ffn_kernel,
        grid_spec=grid_spec,
        out_shape=jax.ShapeDtypeStruct((MAXROWS, D_MODEL), jnp.float32),
        compiler_params=pltpu.CompilerParams(
            dimension_semantics=("arbitrary",)),
    )(te, xs, W1.reshape(NUM_EXPERTS, D_MODEL, HID),
      b1.reshape(NUM_EXPERTS, 1, HID),
      W2.reshape(NUM_EXPERTS, HID, D_MODEL),
      b2.reshape(NUM_EXPERTS, 1, D_MODEL))


def _comb_body(osr_hbm, d0_hbm, d1_hbm, ws_hbm, out_hbm,
               d0_v, d1_v, ws_v, g0_v, g1_v, o_v, sem0, sem1):
    wid = lax.axis_index("s") * NC + lax.axis_index("c")
    base = wid * TPW
    pltpu.sync_copy(d0_hbm.at[wid], d0_v)
    pltpu.sync_copy(d1_hbm.at[wid], d1_v)
    pltpu.sync_copy(ws_hbm.at[pl.ds(base, TPW)], ws_v)
    def sub(j, carry):
        cp0 = pltpu.async_copy(osr_hbm.at[d0_v.at[j]], g0_v, sem0)
        cp1 = pltpu.async_copy(osr_hbm.at[d1_v.at[j]], g1_v, sem1)
        cp0.wait()
        cp1.wait()

        def body(t, _):
            t16 = jnp.full((16,), j * SUB + t, jnp.int32)
            wv = plsc.load_gather(ws_v, [t16])
            for q in range(D_MODEL // 16):
                cs = pl.ds(q * 16, 16)
                o_v[t, cs] = (g0_v[t, cs] + g1_v[t, cs]) * wv
            return 0

        lax.fori_loop(0, SUB, body, 0)
        pltpu.sync_copy(o_v, out_hbm.at[pl.ds(base + j * SUB, SUB)])
        return carry

    lax.fori_loop(0, NSUB, sub, 0)


def _combine(osr, d0, d1, ws):
    mesh = plsc.VectorSubcoreMesh(core_axis_name="c", subcore_axis_name="s")
    f = pl.kernel(
        _comb_body,
        out_type=jax.ShapeDtypeStruct((N_TOK, D_MODEL), jnp.float32),
        mesh=mesh,
        scratch_types=[
            pltpu.VMEM((NSUB, SUB), jnp.int32),
            pltpu.VMEM((NSUB, SUB), jnp.int32),
            pltpu.VMEM((TPW,), jnp.float32),
            pltpu.VMEM((SUB, D_MODEL), jnp.float32),
            pltpu.VMEM((SUB, D_MODEL), jnp.float32),
            pltpu.VMEM((SUB, D_MODEL), jnp.float32),
            pltpu.SemaphoreType.DMA,
            pltpu.SemaphoreType.DMA,
        ],
        compiler_params=pltpu.CompilerParams(needs_layout_passes=False),
    )
    return f(osr, d0, d1, ws)


def kernel(x, Wg, W1, b1, W2, b2):
    e1o, e2o, r0o, r1o, wso, poffo, teo = _gating(x, Wg)
    e1 = e1o.reshape(N_TOK)
    e2 = e2o.reshape(N_TOK)
    r0 = r0o.reshape(N_TOK)
    r1 = r1o.reshape(N_TOK)
    ws = wso.reshape(N_TOK)
    poff = poffo.reshape(128)
    te = teo.reshape(512)
    xs, d0, d1 = _dispatch(x, e1, e2, r0, r1, poff)
    osr = _grouped_ffn(te, xs, W1, b1, W2, b2)
    return _combine(osr, d0, d1, ws)


# speedup vs baseline: 1.6742x; 1.0330x over previous
import functools

import jax
import jax.numpy as jnp
from jax import lax
from jax.experimental import pallas as pl
from jax.experimental.pallas import tpu as pltpu
from jax.experimental.pallas import tpu_sc as plsc

N_TOK = 32768
D_MODEL = 768
NUM_EXPERTS = 8
HID = 384
TOP_K = 2

BLK = 256
NTILES = 2 * N_TOK // BLK + NUM_EXPERTS
MAXROWS = NTILES * BLK
GATE_TILE = 256
NGATE = N_TOK // GATE_TILE
PACKW = D_MODEL // 2

NC, NS = 2, 16
NW = NC * NS
TPW = N_TOK // NW
SUB = 32
NSUB = TPW // SUB

_SQRT1_2 = 0.7071067811865476


def _gate_kernel(x_ref, wg_ref, e1_ref, e2_ref, r0_ref, r1_ref, ws_ref,
                 x16_ref, poff_ref, te_ref, counts):
    i = pl.program_id(0)

    @pl.when(i == 0)
    def _():
        counts[...] = jnp.zeros_like(counts)

    xb = x_ref[...]
    wg = wg_ref[...]
    logits = lax.dot_general(xb, wg, (((1,), (0,)), ((), ())),
                             precision=lax.Precision.DEFAULT)
    lt = logits.T
    iota_e = lax.broadcasted_iota(jnp.int32, (NUM_EXPERTS, GATE_TILE), 0)
    m = jnp.max(lt, axis=0, keepdims=True)
    Z = jnp.sum(jnp.exp(lt - m), axis=0, keepdims=True)
    v1 = m
    e1 = jnp.min(jnp.where(lt == v1, iota_e, NUM_EXPERTS), axis=0,
                 keepdims=True)
    l2 = jnp.where(iota_e == e1, -jnp.inf, lt)
    v2 = jnp.max(l2, axis=0, keepdims=True)
    e2 = jnp.min(jnp.where(l2 == v2, iota_e, NUM_EXPERTS), axis=0,
                 keepdims=True)
    ws = (jnp.exp(v1 - m) + jnp.exp(v2 - m)) / Z

    oh1 = (iota_e == e1).astype(jnp.float32)
    oh2 = (iota_e == e2).astype(jnp.float32)
    oh = oh1 + oh2
    ri = lax.broadcasted_iota(jnp.int32, (GATE_TILE, GATE_TILE), 0)
    ci = lax.broadcasted_iota(jnp.int32, (GATE_TILE, GATE_TILE), 1)
    striu = (ri < ci).astype(jnp.float32)
    cex = lax.dot_general(oh, striu, (((1,), (0,)), ((), ())),
                          precision=lax.Precision.DEFAULT)
    cexc = cex + counts[:, :1]
    r0 = jnp.sum(oh1 * cexc, axis=0, keepdims=True)
    r1 = jnp.sum(oh2 * cexc, axis=0, keepdims=True)
    counts[...] = counts[...] + jnp.sum(oh, axis=1, keepdims=True)

    e1_ref[...] = e1[None]
    e2_ref[...] = e2[None]
    r0_ref[...] = r0.astype(jnp.int32)[None]
    r1_ref[...] = r1.astype(jnp.int32)[None]
    ws_ref[...] = ws[None]
    xlo = xb[:, :PACKW].astype(jnp.bfloat16).astype(jnp.float32)
    xhi = xb[:, PACKW:].astype(jnp.bfloat16).astype(jnp.float32)
    lo = lax.shift_right_logical(
        lax.bitcast_convert_type(xlo, jnp.int32), 16)
    hi = lax.bitcast_convert_type(xhi, jnp.int32) & jnp.int32(-65536)
    x16_ref[...] = lo | hi

    @pl.when(i == NGATE - 1)
    def _():
        cnt = counts[...][:, :1]
        padded = jnp.floor((cnt + (BLK - 1)) / BLK) * BLK
        fi = lax.broadcasted_iota(jnp.int32, (NUM_EXPERTS, 128), 0)
        li = lax.broadcasted_iota(jnp.int32, (NUM_EXPERTS, 128), 1)
        strict = (fi < li).astype(jnp.float32)
        poff = lax.dot_general(padded, strict, (((0,), (0,)), ((), ())),
                               precision=lax.Precision.DEFAULT)
        poffi = poff.astype(jnp.int32)
        poff_ref[...] = poffi
        bt = poffi // BLK
        lane = lax.broadcasted_iota(jnp.int32, (1, 128), 1)
        i512 = lax.broadcasted_iota(jnp.int32, (1, 512), 1)
        s = jnp.zeros((1, 512), jnp.int32)
        for e in range(NUM_EXPERTS):
            be = jnp.sum(jnp.where(lane == e, bt, 0))
            s = s + (i512 >= be).astype(jnp.int32)
        te_ref[...] = jnp.clip(s - 1, 0, NUM_EXPERTS - 1)


def _gating(x, Wg):
    shp = jax.ShapeDtypeStruct
    outs = pl.pallas_call(
        _gate_kernel,
        grid=(NGATE,),
        in_specs=[
            pl.BlockSpec((GATE_TILE, D_MODEL), lambda i: (i, 0)),
            pl.BlockSpec((D_MODEL, NUM_EXPERTS), lambda i: (0, 0)),
        ],
        out_specs=[
            pl.BlockSpec((1, 1, GATE_TILE), lambda i: (i, 0, 0)),
            pl.BlockSpec((1, 1, GATE_TILE), lambda i: (i, 0, 0)),
            pl.BlockSpec((1, 1, GATE_TILE), lambda i: (i, 0, 0)),
            pl.BlockSpec((1, 1, GATE_TILE), lambda i: (i, 0, 0)),
            pl.BlockSpec((1, 1, GATE_TILE), lambda i: (i, 0, 0)),
            pl.BlockSpec((GATE_TILE, PACKW), lambda i: (i, 0)),
            pl.BlockSpec((1, 128), lambda i: (0, 0)),
            pl.BlockSpec((1, 512), lambda i: (0, 0)),
        ],
        out_shape=[
            shp((NGATE, 1, GATE_TILE), jnp.int32),
            shp((NGATE, 1, GATE_TILE), jnp.int32),
            shp((NGATE, 1, GATE_TILE), jnp.int32),
            shp((NGATE, 1, GATE_TILE), jnp.int32),
            shp((NGATE, 1, GATE_TILE), jnp.float32),
            shp((N_TOK, PACKW), jnp.int32),
            shp((1, 128), jnp.int32),
            shp((1, 512), jnp.int32),
        ],
        scratch_shapes=[pltpu.VMEM((NUM_EXPERTS, 128), jnp.float32)],
    )(x, Wg)
    return outs


def _disp_body(x16_hbm, e1_hbm, e2_hbm, r0_hbm, r1_hbm, poff_hbm, ws_hbm,
               xs_hbm, wss_hbm, d0_hbm, d1_hbm,
               poff_v, e1_v, e2_v, r0_v, r1_v, ws_v, idx0_sub, idx1_sub,
               rows_v, sem0, sem1, sem2, sem3):
    wid = lax.axis_index("s") * NC + lax.axis_index("c")
    base = wid * TPW
    pltpu.sync_copy(poff_hbm, poff_v)
    pltpu.sync_copy(e1_hbm.at[pl.ds(base, TPW)], e1_v)
    pltpu.sync_copy(e2_hbm.at[pl.ds(base, TPW)], e2_v)
    pltpu.sync_copy(r0_hbm.at[pl.ds(base, TPW)], r0_v)
    pltpu.sync_copy(r1_hbm.at[pl.ds(base, TPW)], r1_v)
    pltpu.sync_copy(ws_hbm.at[pl.ds(base, TPW)], ws_v)

    def sub(j, carry):
        for k in range(SUB // 16):
            sl = pl.ds(j * SUB + k * 16, 16)
            ko = pl.ds(k * 16, 16)
            idx0_sub[ko] = plsc.load_gather(poff_v, [e1_v[sl]]) + r0_v[sl]
            idx1_sub[ko] = plsc.load_gather(poff_v, [e2_v[sl]]) + r1_v[sl]
        pltpu.sync_copy(idx0_sub, d0_hbm.at[wid, j])
        pltpu.sync_copy(idx1_sub, d1_hbm.at[wid, j])
        pltpu.sync_copy(x16_hbm.at[pl.ds(base + j * SUB, SUB)], rows_v)
        cp0 = pltpu.async_copy(rows_v, xs_hbm.at[idx0_sub], sem0)
        cp1 = pltpu.async_copy(rows_v, xs_hbm.at[idx1_sub], sem1)
        cp2 = pltpu.async_copy(ws_v.at[pl.ds(j * SUB, SUB)],
                               wss_hbm.at[idx0_sub], sem2)
        cp3 = pltpu.async_copy(ws_v.at[pl.ds(j * SUB, SUB)],
                               wss_hbm.at[idx1_sub], sem3)
        cp0.wait()
        cp1.wait()
        cp2.wait()
        cp3.wait()
        return carry

    lax.fori_loop(0, NSUB, sub, 0)


def _dispatch(x16, e1, e2, r0, r1, poff, ws):
    shp = jax.ShapeDtypeStruct
    mesh = plsc.VectorSubcoreMesh(core_axis_name="c", subcore_axis_name="s")
    f = pl.kernel(
        _disp_body,
        out_type=(
            shp((MAXROWS, PACKW), jnp.int32),
            shp((MAXROWS,), jnp.float32),
            shp((NW, NSUB, SUB), jnp.int32),
            shp((NW, NSUB, SUB), jnp.int32),
        ),
        mesh=mesh,
        scratch_types=[
            pltpu.VMEM((128,), jnp.int32),
            pltpu.VMEM((TPW,), jnp.int32),
            pltpu.VMEM((TPW,), jnp.int32),
            pltpu.VMEM((TPW,), jnp.int32),
            pltpu.VMEM((TPW,), jnp.int32),
            pltpu.VMEM((TPW,), jnp.float32),
            pltpu.VMEM((SUB,), jnp.int32),
            pltpu.VMEM((SUB,), jnp.int32),
            pltpu.VMEM((SUB, PACKW), jnp.int32),
            pltpu.SemaphoreType.DMA,
            pltpu.SemaphoreType.DMA,
            pltpu.SemaphoreType.DMA,
            pltpu.SemaphoreType.DMA,
        ],
        compiler_params=pltpu.CompilerParams(needs_layout_passes=False),
    )
    return f(x16, e1, e2, r0, r1, poff, ws)


def _ffn_kernel(te_ref, xs_ref, w1_ref, b1_ref, w2_ref, b2_ref, ws_ref,
                o_ref):
    v = xs_ref[...]
    xlo = lax.bitcast_convert_type(
        lax.shift_left(v, 16), jnp.float32).astype(jnp.bfloat16)
    xhi = lax.bitcast_convert_type(
        v & jnp.int32(-65536), jnp.float32).astype(jnp.bfloat16)
    w1 = w1_ref[0].astype(jnp.bfloat16)
    h = lax.dot_general(xlo, w1[:PACKW], (((1,), (0,)), ((), ())),
                        preferred_element_type=jnp.float32)
    h = h + lax.dot_general(xhi, w1[PACKW:], (((1,), (0,)), ((), ())),
                            preferred_element_type=jnp.float32)
    h = h + b1_ref[0]
    h = 0.5 * h * (1.0 + lax.erf(h * _SQRT1_2))
    hb = h.astype(jnp.bfloat16)
    w2 = w2_ref[0].astype(jnp.bfloat16)
    o = lax.dot_general(hb, w2, (((1,), (0,)), ((), ())),
                        preferred_element_type=jnp.float32)
    o_ref[...] = (o + b2_ref[0]) * ws_ref[...]


def _grouped_ffn(te, xs, W1, b1, W2, b2, wss):
    grid_spec = pltpu.PrefetchScalarGridSpec(
        num_scalar_prefetch=1,
        grid=(NTILES,),
        in_specs=[
            pl.BlockSpec((BLK, PACKW), lambda i, te: (i, 0)),
            pl.BlockSpec((1, D_MODEL, HID), lambda i, te: (te[i], 0, 0)),
            pl.BlockSpec((1, 1, HID), lambda i, te: (te[i], 0, 0)),
            pl.BlockSpec((1, HID, D_MODEL), lambda i, te: (te[i], 0, 0)),
            pl.BlockSpec((1, 1, D_MODEL), lambda i, te: (te[i], 0, 0)),
            pl.BlockSpec((BLK, 1), lambda i, te: (i, 0)),
        ],
        out_specs=pl.BlockSpec((BLK, D_MODEL), lambda i, te: (i, 0)),
    )
    return pl.pallas_call(
        _ffn_kernel,
        grid_spec=grid_spec,
        out_shape=jax.ShapeDtypeStruct((MAXROWS, D_MODEL), jnp.float32),
        compiler_params=pltpu.CompilerParams(
            dimension_semantics=("arbitrary",)),
    )(te, xs, W1.reshape(NUM_EXPERTS, D_MODEL, HID),
      b1.reshape(NUM_EXPERTS, 1, HID),
      W2.reshape(NUM_EXPERTS, HID, D_MODEL),
      b2.reshape(NUM_EXPERTS, 1, D_MODEL),
      wss)


def _comb_body(osr_hbm, d0_hbm, d1_hbm, out_hbm,
               d0_v, d1_v, g0_v, g1_v, o_v, sem0, sem1):
    wid = lax.axis_index("s") * NC + lax.axis_index("c")
    base = wid * TPW
    pltpu.sync_copy(d0_hbm.at[wid], d0_v)
    pltpu.sync_copy(d1_hbm.at[wid], d1_v)
    def sub(j, carry):
        cp0 = pltpu.async_copy(osr_hbm.at[d0_v.at[j]], g0_v, sem0)
        cp1 = pltpu.async_copy(osr_hbm.at[d1_v.at[j]], g1_v, sem1)
        cp0.wait()
        cp1.wait()

        def body(t, _):
            for q in range(D_MODEL // 16):
                cs = pl.ds(q * 16, 16)
                o_v[t, cs] = g0_v[t, cs] + g1_v[t, cs]
            return 0

        lax.fori_loop(0, SUB, body, 0)
        pltpu.sync_copy(o_v, out_hbm.at[pl.ds(base + j * SUB, SUB)])
        return carry

    lax.fori_loop(0, NSUB, sub, 0)


def _combine(osr, d0, d1):
    mesh = plsc.VectorSubcoreMesh(core_axis_name="c", subcore_axis_name="s")
    f = pl.kernel(
        _comb_body,
        out_type=jax.ShapeDtypeStruct((N_TOK, D_MODEL), jnp.float32),
        mesh=mesh,
        scratch_types=[
            pltpu.VMEM((NSUB, SUB), jnp.int32),
            pltpu.VMEM((NSUB, SUB), jnp.int32),
            pltpu.VMEM((SUB, D_MODEL), jnp.float32),
            pltpu.VMEM((SUB, D_MODEL), jnp.float32),
            pltpu.VMEM((SUB, D_MODEL), jnp.float32),
            pltpu.SemaphoreType.DMA,
            pltpu.SemaphoreType.DMA,
        ],
        compiler_params=pltpu.CompilerParams(needs_layout_passes=False),
    )
    return f(osr, d0, d1)


def kernel(x, Wg, W1, b1, W2, b2):
    e1o, e2o, r0o, r1o, wso, x16, poffo, teo = _gating(x, Wg)
    e1 = e1o.reshape(N_TOK)
    e2 = e2o.reshape(N_TOK)
    r0 = r0o.reshape(N_TOK)
    r1 = r1o.reshape(N_TOK)
    ws = wso.reshape(N_TOK)
    poff = poffo.reshape(128)
    te = teo.reshape(512)
    xs, wss, d0, d1 = _dispatch(x16, e1, e2, r0, r1, poff, ws)
    osr = _grouped_ffn(te, xs, W1, b1, W2, b2, wss.reshape(MAXROWS, 1))
    return _combine(osr, d0, d1)


# speedup vs baseline: 1.7626x; 1.0528x over previous
import functools

import jax
import jax.numpy as jnp
from jax import lax
from jax.experimental import pallas as pl
from jax.experimental.pallas import tpu as pltpu
from jax.experimental.pallas import tpu_sc as plsc

N_TOK = 32768
D_MODEL = 768
NUM_EXPERTS = 8
HID = 384
TOP_K = 2

BLK = 256
NTILES = 2 * N_TOK // BLK + NUM_EXPERTS
MAXROWS = NTILES * BLK
GATE_TILE = 256
NGATE = N_TOK // GATE_TILE
PACKW = D_MODEL // 2

NC, NS = 2, 16
NW = NC * NS
TPW = N_TOK // NW
SUB = 32
NSUB = TPW // SUB

_SQRT1_2 = 0.7071067811865476


def _gate_kernel(x_ref, wg_ref, e1_ref, e2_ref, r0_ref, r1_ref, ws_ref,
                 x16_ref, poff_ref, te_ref, counts):
    i = pl.program_id(0)

    @pl.when(i == 0)
    def _():
        counts[...] = jnp.zeros_like(counts)

    xb = x_ref[...]
    wg = wg_ref[...]
    logits = lax.dot_general(xb, wg, (((1,), (0,)), ((), ())),
                             precision=lax.Precision.DEFAULT)
    lt = logits.T
    iota_e = lax.broadcasted_iota(jnp.int32, (NUM_EXPERTS, GATE_TILE), 0)
    m = jnp.max(lt, axis=0, keepdims=True)
    Z = jnp.sum(jnp.exp(lt - m), axis=0, keepdims=True)
    v1 = m
    e1 = jnp.min(jnp.where(lt == v1, iota_e, NUM_EXPERTS), axis=0,
                 keepdims=True)
    l2 = jnp.where(iota_e == e1, -jnp.inf, lt)
    v2 = jnp.max(l2, axis=0, keepdims=True)
    e2 = jnp.min(jnp.where(l2 == v2, iota_e, NUM_EXPERTS), axis=0,
                 keepdims=True)
    ws = (jnp.exp(v1 - m) + jnp.exp(v2 - m)) / Z

    oh1 = (iota_e == e1).astype(jnp.float32)
    oh2 = (iota_e == e2).astype(jnp.float32)
    oh = oh1 + oh2
    ri = lax.broadcasted_iota(jnp.int32, (GATE_TILE, GATE_TILE), 0)
    ci = lax.broadcasted_iota(jnp.int32, (GATE_TILE, GATE_TILE), 1)
    striu = (ri < ci).astype(jnp.float32)
    cex = lax.dot_general(oh, striu, (((1,), (0,)), ((), ())),
                          precision=lax.Precision.DEFAULT)
    cexc = cex + counts[:, :1]
    r0 = jnp.sum(oh1 * cexc, axis=0, keepdims=True)
    r1 = jnp.sum(oh2 * cexc, axis=0, keepdims=True)
    counts[...] = counts[...] + jnp.sum(oh, axis=1, keepdims=True)

    e1_ref[...] = e1[None]
    e2_ref[...] = e2[None]
    r0_ref[...] = r0.astype(jnp.int32)[None]
    r1_ref[...] = r1.astype(jnp.int32)[None]
    ws_ref[...] = ws[None]
    xlo = xb[:, :PACKW].astype(jnp.bfloat16).astype(jnp.float32)
    xhi = xb[:, PACKW:].astype(jnp.bfloat16).astype(jnp.float32)
    lo = lax.shift_right_logical(
        lax.bitcast_convert_type(xlo, jnp.int32), 16)
    hi = lax.bitcast_convert_type(xhi, jnp.int32) & jnp.int32(-65536)
    x16_ref[...] = lo | hi

    @pl.when(i == NGATE - 1)
    def _():
        cnt = counts[...][:, :1]
        padded = jnp.floor((cnt + (BLK - 1)) / BLK) * BLK
        fi = lax.broadcasted_iota(jnp.int32, (NUM_EXPERTS, 128), 0)
        li = lax.broadcasted_iota(jnp.int32, (NUM_EXPERTS, 128), 1)
        strict = (fi < li).astype(jnp.float32)
        poff = lax.dot_general(padded, strict, (((0,), (0,)), ((), ())),
                               precision=lax.Precision.DEFAULT)
        poffi = poff.astype(jnp.int32)
        poff_ref[...] = poffi
        bt = poffi // BLK
        lane = lax.broadcasted_iota(jnp.int32, (1, 128), 1)
        i512 = lax.broadcasted_iota(jnp.int32, (1, 512), 1)
        s = jnp.zeros((1, 512), jnp.int32)
        for e in range(NUM_EXPERTS):
            be = jnp.sum(jnp.where(lane == e, bt, 0))
            s = s + (i512 >= be).astype(jnp.int32)
        te_ref[...] = jnp.clip(s - 1, 0, NUM_EXPERTS - 1)


def _gating(x, Wg):
    shp = jax.ShapeDtypeStruct
    outs = pl.pallas_call(
        _gate_kernel,
        grid=(NGATE,),
        in_specs=[
            pl.BlockSpec((GATE_TILE, D_MODEL), lambda i: (i, 0)),
            pl.BlockSpec((D_MODEL, NUM_EXPERTS), lambda i: (0, 0)),
        ],
        out_specs=[
            pl.BlockSpec((1, 1, GATE_TILE), lambda i: (i, 0, 0)),
            pl.BlockSpec((1, 1, GATE_TILE), lambda i: (i, 0, 0)),
            pl.BlockSpec((1, 1, GATE_TILE), lambda i: (i, 0, 0)),
            pl.BlockSpec((1, 1, GATE_TILE), lambda i: (i, 0, 0)),
            pl.BlockSpec((1, 1, GATE_TILE), lambda i: (i, 0, 0)),
            pl.BlockSpec((GATE_TILE, PACKW), lambda i: (i, 0)),
            pl.BlockSpec((1, 128), lambda i: (0, 0)),
            pl.BlockSpec((1, 512), lambda i: (0, 0)),
        ],
        out_shape=[
            shp((NGATE, 1, GATE_TILE), jnp.int32),
            shp((NGATE, 1, GATE_TILE), jnp.int32),
            shp((NGATE, 1, GATE_TILE), jnp.int32),
            shp((NGATE, 1, GATE_TILE), jnp.int32),
            shp((NGATE, 1, GATE_TILE), jnp.float32),
            shp((N_TOK, PACKW), jnp.int32),
            shp((1, 128), jnp.int32),
            shp((1, 512), jnp.int32),
        ],
        scratch_shapes=[pltpu.VMEM((NUM_EXPERTS, 128), jnp.float32)],
    )(x, Wg)
    return outs


def _disp_body(x16_hbm, e1_hbm, e2_hbm, r0_hbm, r1_hbm, poff_hbm, ws_hbm,
               xs_hbm, wss_hbm, d0_hbm, d1_hbm,
               poff_v, e1_v, e2_v, r0_v, r1_v, ws_v, idx0_sub, idx1_sub,
               wsub, rows2, semL, sem0, sem1, semw0, semw1):
    wid = lax.axis_index("s") * NC + lax.axis_index("c")
    base = wid * TPW
    pltpu.sync_copy(poff_hbm, poff_v)
    pltpu.sync_copy(e1_hbm.at[pl.ds(base, TPW)], e1_v)
    pltpu.sync_copy(e2_hbm.at[pl.ds(base, TPW)], e2_v)
    pltpu.sync_copy(r0_hbm.at[pl.ds(base, TPW)], r0_v)
    pltpu.sync_copy(r1_hbm.at[pl.ds(base, TPW)], r1_v)
    pltpu.sync_copy(ws_hbm.at[pl.ds(base, TPW)], ws_v)
    pltpu.async_copy(x16_hbm.at[pl.ds(base, SUB)], rows2.at[0], semL)

    def sub(j, carry):
        for k in range(SUB // 16):
            sl = pl.ds(j * SUB + k * 16, 16)
            ko = pl.ds(k * 16, 16)
            idx0_sub[ko] = plsc.load_gather(poff_v, [e1_v[sl]]) + r0_v[sl]
            idx1_sub[ko] = plsc.load_gather(poff_v, [e2_v[sl]]) + r1_v[sl]
        pltpu.sync_copy(idx0_sub, d0_hbm.at[wid, j])
        pltpu.sync_copy(idx1_sub, d1_hbm.at[wid, j])
        for k in range(SUB):
            t16 = jnp.full((16,), j * SUB + k, jnp.int32)
            wsub[k, pl.ds(0, 16)] = plsc.load_gather(ws_v, [t16])
        pltpu.make_async_copy(x16_hbm.at[pl.ds(base + j * SUB, SUB)],
                              rows2.at[j % 2], semL).wait()
        jn = (j + 1) % NSUB
        pltpu.async_copy(x16_hbm.at[pl.ds(base + jn * SUB, SUB)],
                         rows2.at[(j + 1) % 2], semL)
        cp0 = pltpu.async_copy(rows2.at[j % 2], xs_hbm.at[idx0_sub], sem0)
        cp1 = pltpu.async_copy(rows2.at[j % 2], xs_hbm.at[idx1_sub], sem1)
        cpw0 = pltpu.async_copy(wsub, wss_hbm.at[idx0_sub], semw0)
        cpw1 = pltpu.async_copy(wsub, wss_hbm.at[idx1_sub], semw1)
        cp0.wait()
        cp1.wait()
        cpw0.wait()
        cpw1.wait()
        return carry

    lax.fori_loop(0, NSUB, sub, 0)
    pltpu.make_async_copy(x16_hbm.at[pl.ds(base, SUB)], rows2.at[0],
                          semL).wait()


def _dispatch(x16, e1, e2, r0, r1, poff, ws):
    shp = jax.ShapeDtypeStruct
    mesh = plsc.VectorSubcoreMesh(core_axis_name="c", subcore_axis_name="s")
    f = pl.kernel(
        _disp_body,
        out_type=(
            shp((MAXROWS, PACKW), jnp.int32),
            shp((MAXROWS, 128), jnp.float32),
            shp((NW, NSUB, SUB), jnp.int32),
            shp((NW, NSUB, SUB), jnp.int32),
        ),
        mesh=mesh,
        scratch_types=[
            pltpu.VMEM((128,), jnp.int32),
            pltpu.VMEM((TPW,), jnp.int32),
            pltpu.VMEM((TPW,), jnp.int32),
            pltpu.VMEM((TPW,), jnp.int32),
            pltpu.VMEM((TPW,), jnp.int32),
            pltpu.VMEM((TPW,), jnp.float32),
            pltpu.VMEM((SUB,), jnp.int32),
            pltpu.VMEM((SUB,), jnp.int32),
            pltpu.VMEM((SUB, 128), jnp.float32),
            pltpu.VMEM((2, SUB, PACKW), jnp.int32),
            pltpu.SemaphoreType.DMA,
            pltpu.SemaphoreType.DMA,
            pltpu.SemaphoreType.DMA,
            pltpu.SemaphoreType.DMA,
            pltpu.SemaphoreType.DMA,
        ],
        compiler_params=pltpu.CompilerParams(needs_layout_passes=False),
    )
    return f(x16, e1, e2, r0, r1, poff, ws)


def _ffn_kernel(te_ref, xs_ref, w1_ref, b1_ref, w2_ref, b2_ref, ws_ref,
                o_ref):
    v = xs_ref[...]
    xlo = lax.bitcast_convert_type(
        lax.shift_left(v, 16), jnp.float32).astype(jnp.bfloat16)
    xhi = lax.bitcast_convert_type(
        v & jnp.int32(-65536), jnp.float32).astype(jnp.bfloat16)
    w1 = w1_ref[0].astype(jnp.bfloat16)
    h = lax.dot_general(xlo, w1[:PACKW], (((1,), (0,)), ((), ())),
                        preferred_element_type=jnp.float32)
    h = h + lax.dot_general(xhi, w1[PACKW:], (((1,), (0,)), ((), ())),
                            preferred_element_type=jnp.float32)
    h = h + b1_ref[0]
    h = 0.5 * h * (1.0 + lax.erf(h * _SQRT1_2))
    hb = h.astype(jnp.bfloat16)
    w2 = w2_ref[0].astype(jnp.bfloat16)
    o = lax.dot_general(hb, w2, (((1,), (0,)), ((), ())),
                        preferred_element_type=jnp.float32)
    o_ref[...] = (o + b2_ref[0]) * ws_ref[:, :1]


def _grouped_ffn(te, xs, W1, b1, W2, b2, wss):
    grid_spec = pltpu.PrefetchScalarGridSpec(
        num_scalar_prefetch=1,
        grid=(NTILES,),
        in_specs=[
            pl.BlockSpec((BLK, PACKW), lambda i, te: (i, 0)),
            pl.BlockSpec((1, D_MODEL, HID), lambda i, te: (te[i], 0, 0)),
            pl.BlockSpec((1, 1, HID), lambda i, te: (te[i], 0, 0)),
            pl.BlockSpec((1, HID, D_MODEL), lambda i, te: (te[i], 0, 0)),
            pl.BlockSpec((1, 1, D_MODEL), lambda i, te: (te[i], 0, 0)),
            pl.BlockSpec((BLK, 128), lambda i, te: (i, 0)),
        ],
        out_specs=pl.BlockSpec((BLK, D_MODEL), lambda i, te: (i, 0)),
    )
    return pl.pallas_call(
        _ffn_kernel,
        grid_spec=grid_spec,
        out_shape=jax.ShapeDtypeStruct((MAXROWS, D_MODEL), jnp.float32),
        compiler_params=pltpu.CompilerParams(
            dimension_semantics=("arbitrary",)),
    )(te, xs, W1.reshape(NUM_EXPERTS, D_MODEL, HID),
      b1.reshape(NUM_EXPERTS, 1, HID),
      W2.reshape(NUM_EXPERTS, HID, D_MODEL),
      b2.reshape(NUM_EXPERTS, 1, D_MODEL),
      wss)


def _comb_body(osr_hbm, d0_hbm, d1_hbm, out_hbm,
               d0_v, d1_v, g0x, g1x, sem0, sem1):
    wid = lax.axis_index("s") * NC + lax.axis_index("c")
    base = wid * TPW
    pltpu.sync_copy(d0_hbm.at[wid], d0_v)
    pltpu.sync_copy(d1_hbm.at[wid], d1_v)
    pltpu.async_copy(osr_hbm.at[d0_v.at[0]], g0x.at[0], sem0)
    pltpu.async_copy(osr_hbm.at[d1_v.at[0]], g1x.at[0], sem1)

    def sub(j, carry):
        s = j % 2
        g0 = g0x.at[s]
        g1 = g1x.at[s]
        pltpu.make_async_copy(osr_hbm.at[d0_v.at[j]], g0, sem0).wait()
        pltpu.make_async_copy(osr_hbm.at[d1_v.at[j]], g1, sem1).wait()
        jn = (j + 1) % NSUB
        sn = (j + 1) % 2
        pltpu.async_copy(osr_hbm.at[d0_v.at[jn]], g0x.at[sn], sem0)
        pltpu.async_copy(osr_hbm.at[d1_v.at[jn]], g1x.at[sn], sem1)

        def body(t, _):
            for q in range(D_MODEL // 16):
                cs = pl.ds(q * 16, 16)
                g0[t, cs] = g0[t, cs] + g1[t, cs]
            return 0

        lax.fori_loop(0, SUB, body, 0)
        pltpu.sync_copy(g0, out_hbm.at[pl.ds(base + j * SUB, SUB)])
        return carry

    lax.fori_loop(0, NSUB, sub, 0)
    pltpu.make_async_copy(osr_hbm.at[d0_v.at[0]], g0x.at[0], sem0).wait()
    pltpu.make_async_copy(osr_hbm.at[d1_v.at[0]], g1x.at[0], sem1).wait()


def _combine(osr, d0, d1):
    mesh = plsc.VectorSubcoreMesh(core_axis_name="c", subcore_axis_name="s")
    f = pl.kernel(
        _comb_body,
        out_type=jax.ShapeDtypeStruct((N_TOK, D_MODEL), jnp.float32),
        mesh=mesh,
        scratch_types=[
            pltpu.VMEM((NSUB, SUB), jnp.int32),
            pltpu.VMEM((NSUB, SUB), jnp.int32),
            pltpu.VMEM((2, SUB, D_MODEL), jnp.float32),
            pltpu.VMEM((2, SUB, D_MODEL), jnp.float32),
            pltpu.SemaphoreType.DMA,
            pltpu.SemaphoreType.DMA,
        ],
        compiler_params=pltpu.CompilerParams(needs_layout_passes=False),
    )
    return f(osr, d0, d1)


def kernel(x, Wg, W1, b1, W2, b2):
    e1o, e2o, r0o, r1o, wso, x16, poffo, teo = _gating(x, Wg)
    e1 = e1o.reshape(N_TOK)
    e2 = e2o.reshape(N_TOK)
    r0 = r0o.reshape(N_TOK)
    r1 = r1o.reshape(N_TOK)
    ws = wso.reshape(N_TOK)
    poff = poffo.reshape(128)
    te = teo.reshape(512)
    xs, wss, d0, d1 = _dispatch(x16, e1, e2, r0, r1, poff, ws)
    osr = _grouped_ffn(te, xs, W1, b1, W2, b2, wss)
    return _combine(osr, d0, d1)


# speedup vs baseline: 2.0181x; 1.1450x over previous
import functools

import jax
import jax.numpy as jnp
from jax import lax
from jax.experimental import pallas as pl
from jax.experimental.pallas import tpu as pltpu
from jax.experimental.pallas import tpu_sc as plsc

N_TOK = 32768
D_MODEL = 768
NUM_EXPERTS = 8
HID = 384
TOP_K = 2

BLK = 256
NTILES = 2 * N_TOK // BLK + NUM_EXPERTS
MAXROWS = NTILES * BLK
GATE_TILE = 256
NGATE = N_TOK // GATE_TILE
PACKW = D_MODEL // 2

NC, NS = 2, 16
NW = NC * NS
TPW = N_TOK // NW
SUB = 32
NSUB = TPW // SUB

_SQRT1_2 = 0.7071067811865476


def _gate_kernel(x_ref, wg_ref, e1_ref, e2_ref, r0_ref, r1_ref, ws_ref,
                 x16_ref, poff_ref, te_ref, counts):
    i = pl.program_id(0)

    @pl.when(i == 0)
    def _():
        counts[...] = jnp.zeros_like(counts)

    xb = x_ref[...]
    wg = wg_ref[...]
    logits = lax.dot_general(xb, wg, (((1,), (0,)), ((), ())),
                             precision=lax.Precision.DEFAULT)
    lt = logits.T
    iota_e = lax.broadcasted_iota(jnp.int32, (NUM_EXPERTS, GATE_TILE), 0)
    m = jnp.max(lt, axis=0, keepdims=True)
    Z = jnp.sum(jnp.exp(lt - m), axis=0, keepdims=True)
    v1 = m
    e1 = jnp.min(jnp.where(lt == v1, iota_e, NUM_EXPERTS), axis=0,
                 keepdims=True)
    l2 = jnp.where(iota_e == e1, -jnp.inf, lt)
    v2 = jnp.max(l2, axis=0, keepdims=True)
    e2 = jnp.min(jnp.where(l2 == v2, iota_e, NUM_EXPERTS), axis=0,
                 keepdims=True)
    ws = (jnp.exp(v1 - m) + jnp.exp(v2 - m)) / Z

    oh1 = (iota_e == e1).astype(jnp.float32)
    oh2 = (iota_e == e2).astype(jnp.float32)
    oh = oh1 + oh2
    ri = lax.broadcasted_iota(jnp.int32, (GATE_TILE, GATE_TILE), 0)
    ci = lax.broadcasted_iota(jnp.int32, (GATE_TILE, GATE_TILE), 1)
    striu = (ri < ci).astype(jnp.float32)
    cex = lax.dot_general(oh, striu, (((1,), (0,)), ((), ())),
                          precision=lax.Precision.DEFAULT)
    cexc = cex + counts[:, :1]
    r0 = jnp.sum(oh1 * cexc, axis=0, keepdims=True)
    r1 = jnp.sum(oh2 * cexc, axis=0, keepdims=True)
    counts[...] = counts[...] + jnp.sum(oh, axis=1, keepdims=True)

    e1_ref[...] = e1[None]
    e2_ref[...] = e2[None]
    r0_ref[...] = r0.astype(jnp.int32)[None]
    r1_ref[...] = r1.astype(jnp.int32)[None]
    ws_ref[...] = ws[None]
    xlo = xb[:, :PACKW].astype(jnp.bfloat16).astype(jnp.float32)
    xhi = xb[:, PACKW:].astype(jnp.bfloat16).astype(jnp.float32)
    lo = lax.shift_right_logical(
        lax.bitcast_convert_type(xlo, jnp.int32), 16)
    hi = lax.bitcast_convert_type(xhi, jnp.int32) & jnp.int32(-65536)
    x16_ref[...] = lo | hi

    @pl.when(i == NGATE - 1)
    def _():
        cnt = counts[...][:, :1]
        padded = jnp.floor((cnt + (BLK - 1)) / BLK) * BLK
        fi = lax.broadcasted_iota(jnp.int32, (NUM_EXPERTS, 128), 0)
        li = lax.broadcasted_iota(jnp.int32, (NUM_EXPERTS, 128), 1)
        strict = (fi < li).astype(jnp.float32)
        poff = lax.dot_general(padded, strict, (((0,), (0,)), ((), ())),
                               precision=lax.Precision.DEFAULT)
        poffi = poff.astype(jnp.int32)
        poff_ref[...] = poffi
        bt = poffi // BLK
        lane = lax.broadcasted_iota(jnp.int32, (1, 128), 1)
        i512 = lax.broadcasted_iota(jnp.int32, (1, 512), 1)
        s = jnp.zeros((1, 512), jnp.int32)
        for e in range(NUM_EXPERTS):
            be = jnp.sum(jnp.where(lane == e, bt, 0))
            s = s + (i512 >= be).astype(jnp.int32)
        te_ref[...] = jnp.clip(s - 1, 0, NUM_EXPERTS - 1)


def _gating(x, Wg):
    shp = jax.ShapeDtypeStruct
    outs = pl.pallas_call(
        _gate_kernel,
        grid=(NGATE,),
        in_specs=[
            pl.BlockSpec((GATE_TILE, D_MODEL), lambda i: (i, 0)),
            pl.BlockSpec((D_MODEL, NUM_EXPERTS), lambda i: (0, 0)),
        ],
        out_specs=[
            pl.BlockSpec((1, 1, GATE_TILE), lambda i: (i, 0, 0)),
            pl.BlockSpec((1, 1, GATE_TILE), lambda i: (i, 0, 0)),
            pl.BlockSpec((1, 1, GATE_TILE), lambda i: (i, 0, 0)),
            pl.BlockSpec((1, 1, GATE_TILE), lambda i: (i, 0, 0)),
            pl.BlockSpec((1, 1, GATE_TILE), lambda i: (i, 0, 0)),
            pl.BlockSpec((GATE_TILE, PACKW), lambda i: (i, 0)),
            pl.BlockSpec((1, 128), lambda i: (0, 0)),
            pl.BlockSpec((1, 512), lambda i: (0, 0)),
        ],
        out_shape=[
            shp((NGATE, 1, GATE_TILE), jnp.int32),
            shp((NGATE, 1, GATE_TILE), jnp.int32),
            shp((NGATE, 1, GATE_TILE), jnp.int32),
            shp((NGATE, 1, GATE_TILE), jnp.int32),
            shp((NGATE, 1, GATE_TILE), jnp.float32),
            shp((N_TOK, PACKW), jnp.int32),
            shp((1, 128), jnp.int32),
            shp((1, 512), jnp.int32),
        ],
        scratch_shapes=[pltpu.VMEM((NUM_EXPERTS, 128), jnp.float32)],
    )(x, Wg)
    return outs


def _disp_body(x16_hbm, e1_hbm, e2_hbm, r0_hbm, r1_hbm, poff_hbm, ws_hbm,
               xs_hbm, wss_hbm, d0_hbm, d1_hbm,
               poff_v, e1_v, e2_v, r0_v, r1_v, ws_v, idx0_sub, idx1_sub,
               wsub, rows2, semL, sem0, sem1, semw0, semw1):
    wid = lax.axis_index("s") * NC + lax.axis_index("c")
    base = wid * TPW
    pltpu.sync_copy(poff_hbm, poff_v)
    pltpu.sync_copy(e1_hbm.at[pl.ds(base, TPW)], e1_v)
    pltpu.sync_copy(e2_hbm.at[pl.ds(base, TPW)], e2_v)
    pltpu.sync_copy(r0_hbm.at[pl.ds(base, TPW)], r0_v)
    pltpu.sync_copy(r1_hbm.at[pl.ds(base, TPW)], r1_v)
    pltpu.sync_copy(ws_hbm.at[pl.ds(base, TPW)], ws_v)
    pltpu.async_copy(x16_hbm.at[pl.ds(base, SUB)], rows2.at[0], semL)

    def sub(j, carry):
        for k in range(SUB // 16):
            sl = pl.ds(j * SUB + k * 16, 16)
            ko = pl.ds(k * 16, 16)
            idx0_sub[ko] = plsc.load_gather(poff_v, [e1_v[sl]]) + r0_v[sl]
            idx1_sub[ko] = plsc.load_gather(poff_v, [e2_v[sl]]) + r1_v[sl]
        pltpu.sync_copy(idx0_sub, d0_hbm.at[wid, j])
        pltpu.sync_copy(idx1_sub, d1_hbm.at[wid, j])
        for k in range(SUB):
            t16 = jnp.full((16,), j * SUB + k, jnp.int32)
            wsub[k, pl.ds(0, 16)] = plsc.load_gather(ws_v, [t16])
        pltpu.make_async_copy(x16_hbm.at[pl.ds(base + j * SUB, SUB)],
                              rows2.at[j % 2], semL).wait()
        jn = (j + 1) % NSUB
        pltpu.async_copy(x16_hbm.at[pl.ds(base + jn * SUB, SUB)],
                         rows2.at[(j + 1) % 2], semL)
        cp0 = pltpu.async_copy(rows2.at[j % 2], xs_hbm.at[idx0_sub], sem0)
        cp1 = pltpu.async_copy(rows2.at[j % 2], xs_hbm.at[idx1_sub], sem1)
        cpw0 = pltpu.async_copy(wsub, wss_hbm.at[idx0_sub], semw0)
        cpw1 = pltpu.async_copy(wsub, wss_hbm.at[idx1_sub], semw1)
        cp0.wait()
        cp1.wait()
        cpw0.wait()
        cpw1.wait()
        return carry

    lax.fori_loop(0, NSUB, sub, 0)
    pltpu.make_async_copy(x16_hbm.at[pl.ds(base, SUB)], rows2.at[0],
                          semL).wait()


def _dispatch(x16, e1, e2, r0, r1, poff, ws):
    shp = jax.ShapeDtypeStruct
    mesh = plsc.VectorSubcoreMesh(core_axis_name="c", subcore_axis_name="s")
    f = pl.kernel(
        _disp_body,
        out_type=(
            shp((MAXROWS, PACKW), jnp.int32),
            shp((MAXROWS, 128), jnp.float32),
            shp((NW, NSUB, SUB), jnp.int32),
            shp((NW, NSUB, SUB), jnp.int32),
        ),
        mesh=mesh,
        scratch_types=[
            pltpu.VMEM((128,), jnp.int32),
            pltpu.VMEM((TPW,), jnp.int32),
            pltpu.VMEM((TPW,), jnp.int32),
            pltpu.VMEM((TPW,), jnp.int32),
            pltpu.VMEM((TPW,), jnp.int32),
            pltpu.VMEM((TPW,), jnp.float32),
            pltpu.VMEM((SUB,), jnp.int32),
            pltpu.VMEM((SUB,), jnp.int32),
            pltpu.VMEM((SUB, 128), jnp.float32),
            pltpu.VMEM((2, SUB, PACKW), jnp.int32),
            pltpu.SemaphoreType.DMA,
            pltpu.SemaphoreType.DMA,
            pltpu.SemaphoreType.DMA,
            pltpu.SemaphoreType.DMA,
            pltpu.SemaphoreType.DMA,
        ],
        compiler_params=pltpu.CompilerParams(needs_layout_passes=False),
    )
    return f(x16, e1, e2, r0, r1, poff, ws)


def _ffn_kernel(te_ref, xs_ref, w1_ref, b1_ref, w2_ref, b2_ref, ws_ref,
                o_ref):
    v = xs_ref[...]
    xlo = lax.bitcast_convert_type(
        lax.shift_left(v, 16), jnp.float32).astype(jnp.bfloat16)
    xhi = lax.bitcast_convert_type(
        v & jnp.int32(-65536), jnp.float32).astype(jnp.bfloat16)
    w1 = w1_ref[0].astype(jnp.bfloat16)
    h = lax.dot_general(xlo, w1[:PACKW], (((1,), (0,)), ((), ())),
                        preferred_element_type=jnp.float32)
    h = h + lax.dot_general(xhi, w1[PACKW:], (((1,), (0,)), ((), ())),
                            preferred_element_type=jnp.float32)
    h = h + b1_ref[0]
    h = 0.5 * h * (1.0 + lax.erf(h * _SQRT1_2))
    hb = h.astype(jnp.bfloat16)
    w2 = w2_ref[0].astype(jnp.bfloat16)
    o = lax.dot_general(hb, w2, (((1,), (0,)), ((), ())),
                        preferred_element_type=jnp.float32)
    of = (o + b2_ref[0]) * ws_ref[:, :1]
    flo = of[:, :PACKW].astype(jnp.bfloat16).astype(jnp.float32)
    fhi = of[:, PACKW:].astype(jnp.bfloat16).astype(jnp.float32)
    lo = lax.shift_right_logical(lax.bitcast_convert_type(flo, jnp.int32), 16)
    hi = lax.bitcast_convert_type(fhi, jnp.int32) & jnp.int32(-65536)
    o_ref[...] = lo | hi


def _grouped_ffn(te, xs, W1, b1, W2, b2, wss):
    grid_spec = pltpu.PrefetchScalarGridSpec(
        num_scalar_prefetch=1,
        grid=(NTILES,),
        in_specs=[
            pl.BlockSpec((BLK, PACKW), lambda i, te: (i, 0)),
            pl.BlockSpec((1, D_MODEL, HID), lambda i, te: (te[i], 0, 0)),
            pl.BlockSpec((1, 1, HID), lambda i, te: (te[i], 0, 0)),
            pl.BlockSpec((1, HID, D_MODEL), lambda i, te: (te[i], 0, 0)),
            pl.BlockSpec((1, 1, D_MODEL), lambda i, te: (te[i], 0, 0)),
            pl.BlockSpec((BLK, 128), lambda i, te: (i, 0)),
        ],
        out_specs=pl.BlockSpec((BLK, PACKW), lambda i, te: (i, 0)),
    )
    return pl.pallas_call(
        _ffn_kernel,
        grid_spec=grid_spec,
        out_shape=jax.ShapeDtypeStruct((MAXROWS, PACKW), jnp.int32),
        compiler_params=pltpu.CompilerParams(
            dimension_semantics=("arbitrary",)),
    )(te, xs, W1.reshape(NUM_EXPERTS, D_MODEL, HID),
      b1.reshape(NUM_EXPERTS, 1, HID),
      W2.reshape(NUM_EXPERTS, HID, D_MODEL),
      b2.reshape(NUM_EXPERTS, 1, D_MODEL),
      wss)


def _comb_body(osr_hbm, d0_hbm, d1_hbm, out_hbm,
               d0_v, d1_v, g0x, g1x, osub, sem0, sem1):
    wid = lax.axis_index("s") * NC + lax.axis_index("c")
    base = wid * TPW
    pltpu.sync_copy(d0_hbm.at[wid], d0_v)
    pltpu.sync_copy(d1_hbm.at[wid], d1_v)
    pltpu.async_copy(osr_hbm.at[d0_v.at[0]], g0x.at[0], sem0)
    pltpu.async_copy(osr_hbm.at[d1_v.at[0]], g1x.at[0], sem1)

    def sub(j, carry):
        s = j % 2
        g0 = g0x.at[s]
        g1 = g1x.at[s]
        pltpu.make_async_copy(osr_hbm.at[d0_v.at[j]], g0, sem0).wait()
        pltpu.make_async_copy(osr_hbm.at[d1_v.at[j]], g1, sem1).wait()
        jn = (j + 1) % NSUB
        sn = (j + 1) % 2
        pltpu.async_copy(osr_hbm.at[d0_v.at[jn]], g0x.at[sn], sem0)
        pltpu.async_copy(osr_hbm.at[d1_v.at[jn]], g1x.at[sn], sem1)

        def body(t, _):
            for q in range(PACKW // 16):
                cs = pl.ds(q * 16, 16)
                a = g0[t, cs]
                b = g1[t, cs]
                osub[t, cs] = (
                    lax.bitcast_convert_type(lax.shift_left(a, 16),
                                             jnp.float32)
                    + lax.bitcast_convert_type(lax.shift_left(b, 16),
                                               jnp.float32))
                osub[t, pl.ds(PACKW + q * 16, 16)] = (
                    lax.bitcast_convert_type(a & jnp.int32(-65536),
                                             jnp.float32)
                    + lax.bitcast_convert_type(b & jnp.int32(-65536),
                                               jnp.float32))
            return 0

        lax.fori_loop(0, SUB, body, 0)
        pltpu.sync_copy(osub, out_hbm.at[pl.ds(base + j * SUB, SUB)])
        return carry

    lax.fori_loop(0, NSUB, sub, 0)
    pltpu.make_async_copy(osr_hbm.at[d0_v.at[0]], g0x.at[0], sem0).wait()
    pltpu.make_async_copy(osr_hbm.at[d1_v.at[0]], g1x.at[0], sem1).wait()


def _combine(osr, d0, d1):
    mesh = plsc.VectorSubcoreMesh(core_axis_name="c", subcore_axis_name="s")
    f = pl.kernel(
        _comb_body,
        out_type=jax.ShapeDtypeStruct((N_TOK, D_MODEL), jnp.float32),
        mesh=mesh,
        scratch_types=[
            pltpu.VMEM((NSUB, SUB), jnp.int32),
            pltpu.VMEM((NSUB, SUB), jnp.int32),
            pltpu.VMEM((2, SUB, PACKW), jnp.int32),
            pltpu.VMEM((2, SUB, PACKW), jnp.int32),
            pltpu.VMEM((SUB, D_MODEL), jnp.float32),
            pltpu.SemaphoreType.DMA,
            pltpu.SemaphoreType.DMA,
        ],
        compiler_params=pltpu.CompilerParams(needs_layout_passes=False),
    )
    return f(osr, d0, d1)


def kernel(x, Wg, W1, b1, W2, b2):
    e1o, e2o, r0o, r1o, wso, x16, poffo, teo = _gating(x, Wg)
    e1 = e1o.reshape(N_TOK)
    e2 = e2o.reshape(N_TOK)
    r0 = r0o.reshape(N_TOK)
    r1 = r1o.reshape(N_TOK)
    ws = wso.reshape(N_TOK)
    poff = poffo.reshape(128)
    te = teo.reshape(512)
    xs, wss, d0, d1 = _dispatch(x16, e1, e2, r0, r1, poff, ws)
    osr = _grouped_ffn(te, xs, W1, b1, W2, b2, wss)
    return _combine(osr, d0, d1)


# speedup vs baseline: 2.2427x; 1.1113x over previous
import functools

import jax
import jax.numpy as jnp
from jax import lax
from jax.experimental import pallas as pl
from jax.experimental.pallas import tpu as pltpu
from jax.experimental.pallas import tpu_sc as plsc

N_TOK = 32768
D_MODEL = 768
NUM_EXPERTS = 8
HID = 384
TOP_K = 2

BLK = 512
NTILES = 2 * N_TOK // BLK + NUM_EXPERTS
MAXROWS = NTILES * BLK
GATE_TILE = 256
NGATE = N_TOK // GATE_TILE
PACKW = D_MODEL // 2

NC, NS = 2, 16
NW = NC * NS
TPW = N_TOK // NW
SUB = 32
NSUB = TPW // SUB

_SQRT1_2 = 0.7071067811865476


def _gate_kernel(x_ref, wg_ref, e1_ref, e2_ref, r0_ref, r1_ref, ws_ref,
                 x16_ref, poff_ref, te_ref, counts):
    i = pl.program_id(0)

    @pl.when(i == 0)
    def _():
        counts[...] = jnp.zeros_like(counts)

    xb = x_ref[...]
    wg = wg_ref[...]
    logits = lax.dot_general(xb, wg, (((1,), (0,)), ((), ())),
                             precision=lax.Precision.DEFAULT)
    lt = logits.T
    iota_e = lax.broadcasted_iota(jnp.int32, (NUM_EXPERTS, GATE_TILE), 0)
    m = jnp.max(lt, axis=0, keepdims=True)
    Z = jnp.sum(jnp.exp(lt - m), axis=0, keepdims=True)
    v1 = m
    e1 = jnp.min(jnp.where(lt == v1, iota_e, NUM_EXPERTS), axis=0,
                 keepdims=True)
    l2 = jnp.where(iota_e == e1, -jnp.inf, lt)
    v2 = jnp.max(l2, axis=0, keepdims=True)
    e2 = jnp.min(jnp.where(l2 == v2, iota_e, NUM_EXPERTS), axis=0,
                 keepdims=True)
    ws = (jnp.exp(v1 - m) + jnp.exp(v2 - m)) / Z

    oh1 = (iota_e == e1).astype(jnp.float32)
    oh2 = (iota_e == e2).astype(jnp.float32)
    oh = oh1 + oh2
    ri = lax.broadcasted_iota(jnp.int32, (GATE_TILE, GATE_TILE), 0)
    ci = lax.broadcasted_iota(jnp.int32, (GATE_TILE, GATE_TILE), 1)
    striu = (ri < ci).astype(jnp.float32)
    cex = lax.dot_general(oh, striu, (((1,), (0,)), ((), ())),
                          precision=lax.Precision.DEFAULT)
    cexc = cex + counts[:, :1]
    r0 = jnp.sum(oh1 * cexc, axis=0, keepdims=True)
    r1 = jnp.sum(oh2 * cexc, axis=0, keepdims=True)
    counts[...] = counts[...] + jnp.sum(oh, axis=1, keepdims=True)

    e1_ref[...] = e1[None]
    e2_ref[...] = e2[None]
    r0_ref[...] = r0.astype(jnp.int32)[None]
    r1_ref[...] = r1.astype(jnp.int32)[None]
    ws_ref[...] = ws[None]
    xlo = xb[:, :PACKW].astype(jnp.bfloat16).astype(jnp.float32)
    xhi = xb[:, PACKW:].astype(jnp.bfloat16).astype(jnp.float32)
    lo = lax.shift_right_logical(
        lax.bitcast_convert_type(xlo, jnp.int32), 16)
    hi = lax.bitcast_convert_type(xhi, jnp.int32) & jnp.int32(-65536)
    x16_ref[...] = lo | hi

    @pl.when(i == NGATE - 1)
    def _():
        cnt = counts[...][:, :1]
        padded = jnp.floor((cnt + (BLK - 1)) / BLK) * BLK
        fi = lax.broadcasted_iota(jnp.int32, (NUM_EXPERTS, 128), 0)
        li = lax.broadcasted_iota(jnp.int32, (NUM_EXPERTS, 128), 1)
        strict = (fi < li).astype(jnp.float32)
        poff = lax.dot_general(padded, strict, (((0,), (0,)), ((), ())),
                               precision=lax.Precision.DEFAULT)
        poffi = poff.astype(jnp.int32)
        poff_ref[...] = poffi
        bt = poffi // BLK
        lane = lax.broadcasted_iota(jnp.int32, (1, 128), 1)
        i512 = lax.broadcasted_iota(jnp.int32, (1, 512), 1)
        s = jnp.zeros((1, 512), jnp.int32)
        for e in range(NUM_EXPERTS):
            be = jnp.sum(jnp.where(lane == e, bt, 0))
            s = s + (i512 >= be).astype(jnp.int32)
        te_ref[...] = jnp.clip(s - 1, 0, NUM_EXPERTS - 1)


def _gating(x, Wg):
    shp = jax.ShapeDtypeStruct
    outs = pl.pallas_call(
        _gate_kernel,
        grid=(NGATE,),
        in_specs=[
            pl.BlockSpec((GATE_TILE, D_MODEL), lambda i: (i, 0)),
            pl.BlockSpec((D_MODEL, NUM_EXPERTS), lambda i: (0, 0)),
        ],
        out_specs=[
            pl.BlockSpec((1, 1, GATE_TILE), lambda i: (i, 0, 0)),
            pl.BlockSpec((1, 1, GATE_TILE), lambda i: (i, 0, 0)),
            pl.BlockSpec((1, 1, GATE_TILE), lambda i: (i, 0, 0)),
            pl.BlockSpec((1, 1, GATE_TILE), lambda i: (i, 0, 0)),
            pl.BlockSpec((1, 1, GATE_TILE), lambda i: (i, 0, 0)),
            pl.BlockSpec((GATE_TILE, PACKW), lambda i: (i, 0)),
            pl.BlockSpec((1, 128), lambda i: (0, 0)),
            pl.BlockSpec((1, 512), lambda i: (0, 0)),
        ],
        out_shape=[
            shp((NGATE, 1, GATE_TILE), jnp.int32),
            shp((NGATE, 1, GATE_TILE), jnp.int32),
            shp((NGATE, 1, GATE_TILE), jnp.int32),
            shp((NGATE, 1, GATE_TILE), jnp.int32),
            shp((NGATE, 1, GATE_TILE), jnp.float32),
            shp((N_TOK, PACKW), jnp.int32),
            shp((1, 128), jnp.int32),
            shp((1, 512), jnp.int32),
        ],
        scratch_shapes=[pltpu.VMEM((NUM_EXPERTS, 128), jnp.float32)],
    )(x, Wg)
    return outs


def _disp_body(x16_hbm, e1_hbm, e2_hbm, r0_hbm, r1_hbm, poff_hbm, ws_hbm,
               xs_hbm, wss_hbm, d0_hbm, d1_hbm,
               poff_v, e1_v, e2_v, r0_v, r1_v, ws_v, idx0_sub, idx1_sub,
               wsub, rows2, semL, sem0, sem1, semw0, semw1):
    wid = lax.axis_index("s") * NC + lax.axis_index("c")
    base = wid * TPW
    pltpu.sync_copy(poff_hbm, poff_v)
    pltpu.sync_copy(e1_hbm.at[pl.ds(base, TPW)], e1_v)
    pltpu.sync_copy(e2_hbm.at[pl.ds(base, TPW)], e2_v)
    pltpu.sync_copy(r0_hbm.at[pl.ds(base, TPW)], r0_v)
    pltpu.sync_copy(r1_hbm.at[pl.ds(base, TPW)], r1_v)
    pltpu.sync_copy(ws_hbm.at[pl.ds(base, TPW)], ws_v)
    pltpu.async_copy(x16_hbm.at[pl.ds(base, SUB)], rows2.at[0], semL)

    def sub(j, carry):
        for k in range(SUB // 16):
            sl = pl.ds(j * SUB + k * 16, 16)
            ko = pl.ds(k * 16, 16)
            idx0_sub[ko] = plsc.load_gather(poff_v, [e1_v[sl]]) + r0_v[sl]
            idx1_sub[ko] = plsc.load_gather(poff_v, [e2_v[sl]]) + r1_v[sl]
        pltpu.sync_copy(idx0_sub, d0_hbm.at[wid, j])
        pltpu.sync_copy(idx1_sub, d1_hbm.at[wid, j])
        for k in range(SUB):
            t16 = jnp.full((16,), j * SUB + k, jnp.int32)
            wsub[k, pl.ds(0, 16)] = plsc.load_gather(ws_v, [t16])
        pltpu.make_async_copy(x16_hbm.at[pl.ds(base + j * SUB, SUB)],
                              rows2.at[j % 2], semL).wait()
        jn = (j + 1) % NSUB
        pltpu.async_copy(x16_hbm.at[pl.ds(base + jn * SUB, SUB)],
                         rows2.at[(j + 1) % 2], semL)
        cp0 = pltpu.async_copy(rows2.at[j % 2], xs_hbm.at[idx0_sub], sem0)
        cp1 = pltpu.async_copy(rows2.at[j % 2], xs_hbm.at[idx1_sub], sem1)
        cpw0 = pltpu.async_copy(wsub, wss_hbm.at[idx0_sub], semw0)
        cpw1 = pltpu.async_copy(wsub, wss_hbm.at[idx1_sub], semw1)
        cp0.wait()
        cp1.wait()
        cpw0.wait()
        cpw1.wait()
        return carry

    lax.fori_loop(0, NSUB, sub, 0)
    pltpu.make_async_copy(x16_hbm.at[pl.ds(base, SUB)], rows2.at[0],
                          semL).wait()


def _dispatch(x16, e1, e2, r0, r1, poff, ws):
    shp = jax.ShapeDtypeStruct
    mesh = plsc.VectorSubcoreMesh(core_axis_name="c", subcore_axis_name="s")
    f = pl.kernel(
        _disp_body,
        out_type=(
            shp((MAXROWS, PACKW), jnp.int32),
            shp((MAXROWS, 128), jnp.float32),
            shp((NW, NSUB, SUB), jnp.int32),
            shp((NW, NSUB, SUB), jnp.int32),
        ),
        mesh=mesh,
        scratch_types=[
            pltpu.VMEM((128,), jnp.int32),
            pltpu.VMEM((TPW,), jnp.int32),
            pltpu.VMEM((TPW,), jnp.int32),
            pltpu.VMEM((TPW,), jnp.int32),
            pltpu.VMEM((TPW,), jnp.int32),
            pltpu.VMEM((TPW,), jnp.float32),
            pltpu.VMEM((SUB,), jnp.int32),
            pltpu.VMEM((SUB,), jnp.int32),
            pltpu.VMEM((SUB, 128), jnp.float32),
            pltpu.VMEM((2, SUB, PACKW), jnp.int32),
            pltpu.SemaphoreType.DMA,
            pltpu.SemaphoreType.DMA,
            pltpu.SemaphoreType.DMA,
            pltpu.SemaphoreType.DMA,
            pltpu.SemaphoreType.DMA,
        ],
        compiler_params=pltpu.CompilerParams(needs_layout_passes=False),
    )
    return f(x16, e1, e2, r0, r1, poff, ws)


def _ffn_kernel(te_ref, xs_ref, w1_ref, b1_ref, w2_ref, b2_ref, ws_ref,
                o_ref):
    v = xs_ref[...]
    xlo = lax.bitcast_convert_type(
        lax.shift_left(v, 16), jnp.float32).astype(jnp.bfloat16)
    xhi = lax.bitcast_convert_type(
        v & jnp.int32(-65536), jnp.float32).astype(jnp.bfloat16)
    w1 = w1_ref[0].astype(jnp.bfloat16)
    h = lax.dot_general(xlo, w1[:PACKW], (((1,), (0,)), ((), ())),
                        preferred_element_type=jnp.float32)
    h = h + lax.dot_general(xhi, w1[PACKW:], (((1,), (0,)), ((), ())),
                            preferred_element_type=jnp.float32)
    h = h + b1_ref[0]
    h = 0.5 * h * (1.0 + lax.erf(h * _SQRT1_2))
    hb = h.astype(jnp.bfloat16)
    w2 = w2_ref[0].astype(jnp.bfloat16)
    o = lax.dot_general(hb, w2, (((1,), (0,)), ((), ())),
                        preferred_element_type=jnp.float32)
    of = (o + b2_ref[0]) * ws_ref[:, :1]
    flo = of[:, :PACKW].astype(jnp.bfloat16).astype(jnp.float32)
    fhi = of[:, PACKW:].astype(jnp.bfloat16).astype(jnp.float32)
    lo = lax.shift_right_logical(lax.bitcast_convert_type(flo, jnp.int32), 16)
    hi = lax.bitcast_convert_type(fhi, jnp.int32) & jnp.int32(-65536)
    o_ref[...] = lo | hi


def _grouped_ffn(te, xs, W1, b1, W2, b2, wss):
    grid_spec = pltpu.PrefetchScalarGridSpec(
        num_scalar_prefetch=1,
        grid=(NTILES,),
        in_specs=[
            pl.BlockSpec((BLK, PACKW), lambda i, te: (i, 0)),
            pl.BlockSpec((1, D_MODEL, HID), lambda i, te: (te[i], 0, 0)),
            pl.BlockSpec((1, 1, HID), lambda i, te: (te[i], 0, 0)),
            pl.BlockSpec((1, HID, D_MODEL), lambda i, te: (te[i], 0, 0)),
            pl.BlockSpec((1, 1, D_MODEL), lambda i, te: (te[i], 0, 0)),
            pl.BlockSpec((BLK, 128), lambda i, te: (i, 0)),
        ],
        out_specs=pl.BlockSpec((BLK, PACKW), lambda i, te: (i, 0)),
    )
    return pl.pallas_call(
        _ffn_kernel,
        grid_spec=grid_spec,
        out_shape=jax.ShapeDtypeStruct((MAXROWS, PACKW), jnp.int32),
        compiler_params=pltpu.CompilerParams(
            dimension_semantics=("arbitrary",)),
    )(te, xs, W1.reshape(NUM_EXPERTS, D_MODEL, HID),
      b1.reshape(NUM_EXPERTS, 1, HID),
      W2.reshape(NUM_EXPERTS, HID, D_MODEL),
      b2.reshape(NUM_EXPERTS, 1, D_MODEL),
      wss)


def _comb_body(osr_hbm, d0_hbm, d1_hbm, out_hbm,
               d0_v, d1_v, g0x, g1x, osub, sem0, sem1):
    wid = lax.axis_index("s") * NC + lax.axis_index("c")
    base = wid * TPW
    pltpu.sync_copy(d0_hbm.at[wid], d0_v)
    pltpu.sync_copy(d1_hbm.at[wid], d1_v)
    pltpu.async_copy(osr_hbm.at[d0_v.at[0]], g0x.at[0], sem0)
    pltpu.async_copy(osr_hbm.at[d1_v.at[0]], g1x.at[0], sem1)

    def sub(j, carry):
        s = j % 2
        g0 = g0x.at[s]
        g1 = g1x.at[s]
        pltpu.make_async_copy(osr_hbm.at[d0_v.at[j]], g0, sem0).wait()
        pltpu.make_async_copy(osr_hbm.at[d1_v.at[j]], g1, sem1).wait()
        jn = (j + 1) % NSUB
        sn = (j + 1) % 2
        pltpu.async_copy(osr_hbm.at[d0_v.at[jn]], g0x.at[sn], sem0)
        pltpu.async_copy(osr_hbm.at[d1_v.at[jn]], g1x.at[sn], sem1)

        def body(t, _):
            for q in range(PACKW // 16):
                cs = pl.ds(q * 16, 16)
                a = g0[t, cs]
                b = g1[t, cs]
                osub[t, cs] = (
                    lax.bitcast_convert_type(lax.shift_left(a, 16),
                                             jnp.float32)
                    + lax.bitcast_convert_type(lax.shift_left(b, 16),
                                               jnp.float32))
                osub[t, pl.ds(PACKW + q * 16, 16)] = (
                    lax.bitcast_convert_type(a & jnp.int32(-65536),
                                             jnp.float32)
                    + lax.bitcast_convert_type(b & jnp.int32(-65536),
                                               jnp.float32))
            return 0

        lax.fori_loop(0, SUB, body, 0)
        pltpu.sync_copy(osub, out_hbm.at[pl.ds(base + j * SUB, SUB)])
        return carry

    lax.fori_loop(0, NSUB, sub, 0)
    pltpu.make_async_copy(osr_hbm.at[d0_v.at[0]], g0x.at[0], sem0).wait()
    pltpu.make_async_copy(osr_hbm.at[d1_v.at[0]], g1x.at[0], sem1).wait()


def _combine(osr, d0, d1):
    mesh = plsc.VectorSubcoreMesh(core_axis_name="c", subcore_axis_name="s")
    f = pl.kernel(
        _comb_body,
        out_type=jax.ShapeDtypeStruct((N_TOK, D_MODEL), jnp.float32),
        mesh=mesh,
        scratch_types=[
            pltpu.VMEM((NSUB, SUB), jnp.int32),
            pltpu.VMEM((NSUB, SUB), jnp.int32),
            pltpu.VMEM((2, SUB, PACKW), jnp.int32),
            pltpu.VMEM((2, SUB, PACKW), jnp.int32),
            pltpu.VMEM((SUB, D_MODEL), jnp.float32),
            pltpu.SemaphoreType.DMA,
            pltpu.SemaphoreType.DMA,
        ],
        compiler_params=pltpu.CompilerParams(needs_layout_passes=False),
    )
    return f(osr, d0, d1)


def kernel(x, Wg, W1, b1, W2, b2):
    e1o, e2o, r0o, r1o, wso, x16, poffo, teo = _gating(x, Wg)
    e1 = e1o.reshape(N_TOK)
    e2 = e2o.reshape(N_TOK)
    r0 = r0o.reshape(N_TOK)
    r1 = r1o.reshape(N_TOK)
    ws = wso.reshape(N_TOK)
    poff = poffo.reshape(128)
    te = teo.reshape(512)
    xs, wss, d0, d1 = _dispatch(x16, e1, e2, r0, r1, poff, ws)
    osr = _grouped_ffn(te, xs, W1, b1, W2, b2, wss)
    return _combine(osr, d0, d1)


# speedup vs baseline: 2.3338x; 1.0407x over previous
import functools

import jax
import jax.numpy as jnp
from jax import lax
from jax.experimental import pallas as pl
from jax.experimental.pallas import tpu as pltpu
from jax.experimental.pallas import tpu_sc as plsc

N_TOK = 32768
D_MODEL = 768
NUM_EXPERTS = 8
HID = 384
TOP_K = 2

BLK = 1024
NTILES = 2 * N_TOK // BLK + NUM_EXPERTS
MAXROWS = NTILES * BLK
GATE_TILE = 256
NGATE = N_TOK // GATE_TILE
PACKW = D_MODEL // 2

NC, NS = 2, 16
NW = NC * NS
TPW = N_TOK // NW
SUB = 32
NSUB = TPW // SUB

_SQRT1_2 = 0.7071067811865476


def _gate_kernel(x_ref, wg_ref, e1_ref, e2_ref, r0_ref, r1_ref, ws_ref,
                 x16_ref, poff_ref, te_ref, counts):
    i = pl.program_id(0)

    @pl.when(i == 0)
    def _():
        counts[...] = jnp.zeros_like(counts)

    xb = x_ref[...]
    wg = wg_ref[...]
    logits = lax.dot_general(xb, wg, (((1,), (0,)), ((), ())),
                             precision=lax.Precision.DEFAULT)
    lt = logits.T
    iota_e = lax.broadcasted_iota(jnp.int32, (NUM_EXPERTS, GATE_TILE), 0)
    m = jnp.max(lt, axis=0, keepdims=True)
    Z = jnp.sum(jnp.exp(lt - m), axis=0, keepdims=True)
    v1 = m
    e1 = jnp.min(jnp.where(lt == v1, iota_e, NUM_EXPERTS), axis=0,
                 keepdims=True)
    l2 = jnp.where(iota_e == e1, -jnp.inf, lt)
    v2 = jnp.max(l2, axis=0, keepdims=True)
    e2 = jnp.min(jnp.where(l2 == v2, iota_e, NUM_EXPERTS), axis=0,
                 keepdims=True)
    ws = (jnp.exp(v1 - m) + jnp.exp(v2 - m)) / Z

    oh1 = (iota_e == e1).astype(jnp.float32)
    oh2 = (iota_e == e2).astype(jnp.float32)
    oh = oh1 + oh2
    ri = lax.broadcasted_iota(jnp.int32, (GATE_TILE, GATE_TILE), 0)
    ci = lax.broadcasted_iota(jnp.int32, (GATE_TILE, GATE_TILE), 1)
    striu = (ri < ci).astype(jnp.float32)
    cex = lax.dot_general(oh, striu, (((1,), (0,)), ((), ())),
                          precision=lax.Precision.DEFAULT)
    cexc = cex + counts[:, :1]
    r0 = jnp.sum(oh1 * cexc, axis=0, keepdims=True)
    r1 = jnp.sum(oh2 * cexc, axis=0, keepdims=True)
    counts[...] = counts[...] + jnp.sum(oh, axis=1, keepdims=True)

    e1_ref[...] = e1[None]
    e2_ref[...] = e2[None]
    r0_ref[...] = r0.astype(jnp.int32)[None]
    r1_ref[...] = r1.astype(jnp.int32)[None]
    ws_ref[...] = ws[None]
    xlo = xb[:, :PACKW].astype(jnp.bfloat16).astype(jnp.float32)
    xhi = xb[:, PACKW:].astype(jnp.bfloat16).astype(jnp.float32)
    lo = lax.shift_right_logical(
        lax.bitcast_convert_type(xlo, jnp.int32), 16)
    hi = lax.bitcast_convert_type(xhi, jnp.int32) & jnp.int32(-65536)
    x16_ref[...] = lo | hi

    @pl.when(i == NGATE - 1)
    def _():
        cnt = counts[...][:, :1]
        padded = jnp.floor((cnt + (BLK - 1)) / BLK) * BLK
        fi = lax.broadcasted_iota(jnp.int32, (NUM_EXPERTS, 128), 0)
        li = lax.broadcasted_iota(jnp.int32, (NUM_EXPERTS, 128), 1)
        strict = (fi < li).astype(jnp.float32)
        poff = lax.dot_general(padded, strict, (((0,), (0,)), ((), ())),
                               precision=lax.Precision.DEFAULT)
        poffi = poff.astype(jnp.int32)
        poff_ref[...] = poffi
        bt = poffi // BLK
        lane = lax.broadcasted_iota(jnp.int32, (1, 128), 1)
        i512 = lax.broadcasted_iota(jnp.int32, (1, 512), 1)
        s = jnp.zeros((1, 512), jnp.int32)
        for e in range(NUM_EXPERTS):
            be = jnp.sum(jnp.where(lane == e, bt, 0))
            s = s + (i512 >= be).astype(jnp.int32)
        te_ref[...] = jnp.clip(s - 1, 0, NUM_EXPERTS - 1)


def _gating(x, Wg):
    shp = jax.ShapeDtypeStruct
    outs = pl.pallas_call(
        _gate_kernel,
        grid=(NGATE,),
        in_specs=[
            pl.BlockSpec((GATE_TILE, D_MODEL), lambda i: (i, 0)),
            pl.BlockSpec((D_MODEL, NUM_EXPERTS), lambda i: (0, 0)),
        ],
        out_specs=[
            pl.BlockSpec((1, 1, GATE_TILE), lambda i: (i, 0, 0)),
            pl.BlockSpec((1, 1, GATE_TILE), lambda i: (i, 0, 0)),
            pl.BlockSpec((1, 1, GATE_TILE), lambda i: (i, 0, 0)),
            pl.BlockSpec((1, 1, GATE_TILE), lambda i: (i, 0, 0)),
            pl.BlockSpec((1, 1, GATE_TILE), lambda i: (i, 0, 0)),
            pl.BlockSpec((GATE_TILE, PACKW), lambda i: (i, 0)),
            pl.BlockSpec((1, 128), lambda i: (0, 0)),
            pl.BlockSpec((1, 512), lambda i: (0, 0)),
        ],
        out_shape=[
            shp((NGATE, 1, GATE_TILE), jnp.int32),
            shp((NGATE, 1, GATE_TILE), jnp.int32),
            shp((NGATE, 1, GATE_TILE), jnp.int32),
            shp((NGATE, 1, GATE_TILE), jnp.int32),
            shp((NGATE, 1, GATE_TILE), jnp.float32),
            shp((N_TOK, PACKW), jnp.int32),
            shp((1, 128), jnp.int32),
            shp((1, 512), jnp.int32),
        ],
        scratch_shapes=[pltpu.VMEM((NUM_EXPERTS, 128), jnp.float32)],
    )(x, Wg)
    return outs


def _disp_body(x16_hbm, e1_hbm, e2_hbm, r0_hbm, r1_hbm, poff_hbm, ws_hbm,
               xs_hbm, wss_hbm, d0_hbm, d1_hbm,
               poff_v, e1_v, e2_v, r0_v, r1_v, ws_v, idx0_sub, idx1_sub,
               wsub, rows2, semL, sem0, sem1, semw0, semw1):
    wid = lax.axis_index("s") * NC + lax.axis_index("c")
    base = wid * TPW
    pltpu.sync_copy(poff_hbm, poff_v)
    pltpu.sync_copy(e1_hbm.at[pl.ds(base, TPW)], e1_v)
    pltpu.sync_copy(e2_hbm.at[pl.ds(base, TPW)], e2_v)
    pltpu.sync_copy(r0_hbm.at[pl.ds(base, TPW)], r0_v)
    pltpu.sync_copy(r1_hbm.at[pl.ds(base, TPW)], r1_v)
    pltpu.sync_copy(ws_hbm.at[pl.ds(base, TPW)], ws_v)
    pltpu.async_copy(x16_hbm.at[pl.ds(base, SUB)], rows2.at[0], semL)

    def sub(j, carry):
        for k in range(SUB // 16):
            sl = pl.ds(j * SUB + k * 16, 16)
            ko = pl.ds(k * 16, 16)
            idx0_sub[ko] = plsc.load_gather(poff_v, [e1_v[sl]]) + r0_v[sl]
            idx1_sub[ko] = plsc.load_gather(poff_v, [e2_v[sl]]) + r1_v[sl]
        pltpu.sync_copy(idx0_sub, d0_hbm.at[wid, j])
        pltpu.sync_copy(idx1_sub, d1_hbm.at[wid, j])
        for k in range(SUB):
            t16 = jnp.full((16,), j * SUB + k, jnp.int32)
            wsub[k, pl.ds(0, 16)] = plsc.load_gather(ws_v, [t16])
        pltpu.make_async_copy(x16_hbm.at[pl.ds(base + j * SUB, SUB)],
                              rows2.at[j % 2], semL).wait()
        jn = (j + 1) % NSUB
        pltpu.async_copy(x16_hbm.at[pl.ds(base + jn * SUB, SUB)],
                         rows2.at[(j + 1) % 2], semL)
        cp0 = pltpu.async_copy(rows2.at[j % 2], xs_hbm.at[idx0_sub], sem0)
        cp1 = pltpu.async_copy(rows2.at[j % 2], xs_hbm.at[idx1_sub], sem1)
        cpw0 = pltpu.async_copy(wsub, wss_hbm.at[idx0_sub], semw0)
        cpw1 = pltpu.async_copy(wsub, wss_hbm.at[idx1_sub], semw1)
        cp0.wait()
        cp1.wait()
        cpw0.wait()
        cpw1.wait()
        return carry

    lax.fori_loop(0, NSUB, sub, 0)
    pltpu.make_async_copy(x16_hbm.at[pl.ds(base, SUB)], rows2.at[0],
                          semL).wait()


def _dispatch(x16, e1, e2, r0, r1, poff, ws):
    shp = jax.ShapeDtypeStruct
    mesh = plsc.VectorSubcoreMesh(core_axis_name="c", subcore_axis_name="s")
    f = pl.kernel(
        _disp_body,
        out_type=(
            shp((MAXROWS, PACKW), jnp.int32),
            shp((MAXROWS, 128), jnp.float32),
            shp((NW, NSUB, SUB), jnp.int32),
            shp((NW, NSUB, SUB), jnp.int32),
        ),
        mesh=mesh,
        scratch_types=[
            pltpu.VMEM((128,), jnp.int32),
            pltpu.VMEM((TPW,), jnp.int32),
            pltpu.VMEM((TPW,), jnp.int32),
            pltpu.VMEM((TPW,), jnp.int32),
            pltpu.VMEM((TPW,), jnp.int32),
            pltpu.VMEM((TPW,), jnp.float32),
            pltpu.VMEM((SUB,), jnp.int32),
            pltpu.VMEM((SUB,), jnp.int32),
            pltpu.VMEM((SUB, 128), jnp.float32),
            pltpu.VMEM((2, SUB, PACKW), jnp.int32),
            pltpu.SemaphoreType.DMA,
            pltpu.SemaphoreType.DMA,
            pltpu.SemaphoreType.DMA,
            pltpu.SemaphoreType.DMA,
            pltpu.SemaphoreType.DMA,
        ],
        compiler_params=pltpu.CompilerParams(needs_layout_passes=False),
    )
    return f(x16, e1, e2, r0, r1, poff, ws)


def _ffn_kernel(te_ref, xs_ref, w1_ref, b1_ref, w2_ref, b2_ref, ws_ref,
                o_ref):
    v = xs_ref[...]
    xlo = lax.bitcast_convert_type(
        lax.shift_left(v, 16), jnp.float32).astype(jnp.bfloat16)
    xhi = lax.bitcast_convert_type(
        v & jnp.int32(-65536), jnp.float32).astype(jnp.bfloat16)
    w1 = w1_ref[0].astype(jnp.bfloat16)
    h = lax.dot_general(xlo, w1[:PACKW], (((1,), (0,)), ((), ())),
                        preferred_element_type=jnp.float32)
    h = h + lax.dot_general(xhi, w1[PACKW:], (((1,), (0,)), ((), ())),
                            preferred_element_type=jnp.float32)
    h = h + b1_ref[0]
    h = 0.5 * h * (1.0 + lax.erf(h * _SQRT1_2))
    hb = h.astype(jnp.bfloat16)
    w2 = w2_ref[0].astype(jnp.bfloat16)
    o = lax.dot_general(hb, w2, (((1,), (0,)), ((), ())),
                        preferred_element_type=jnp.float32)
    of = (o + b2_ref[0]) * ws_ref[:, :1]
    flo = of[:, :PACKW].astype(jnp.bfloat16).astype(jnp.float32)
    fhi = of[:, PACKW:].astype(jnp.bfloat16).astype(jnp.float32)
    lo = lax.shift_right_logical(lax.bitcast_convert_type(flo, jnp.int32), 16)
    hi = lax.bitcast_convert_type(fhi, jnp.int32) & jnp.int32(-65536)
    o_ref[...] = lo | hi


def _grouped_ffn(te, xs, W1, b1, W2, b2, wss):
    grid_spec = pltpu.PrefetchScalarGridSpec(
        num_scalar_prefetch=1,
        grid=(NTILES,),
        in_specs=[
            pl.BlockSpec((BLK, PACKW), lambda i, te: (i, 0)),
            pl.BlockSpec((1, D_MODEL, HID), lambda i, te: (te[i], 0, 0)),
            pl.BlockSpec((1, 1, HID), lambda i, te: (te[i], 0, 0)),
            pl.BlockSpec((1, HID, D_MODEL), lambda i, te: (te[i], 0, 0)),
            pl.BlockSpec((1, 1, D_MODEL), lambda i, te: (te[i], 0, 0)),
            pl.BlockSpec((BLK, 128), lambda i, te: (i, 0)),
        ],
        out_specs=pl.BlockSpec((BLK, PACKW), lambda i, te: (i, 0)),
    )
    return pl.pallas_call(
        _ffn_kernel,
        grid_spec=grid_spec,
        out_shape=jax.ShapeDtypeStruct((MAXROWS, PACKW), jnp.int32),
        compiler_params=pltpu.CompilerParams(
            dimension_semantics=("arbitrary",)),
    )(te, xs, W1.reshape(NUM_EXPERTS, D_MODEL, HID),
      b1.reshape(NUM_EXPERTS, 1, HID),
      W2.reshape(NUM_EXPERTS, HID, D_MODEL),
      b2.reshape(NUM_EXPERTS, 1, D_MODEL),
      wss)


def _comb_body(osr_hbm, d0_hbm, d1_hbm, out_hbm,
               d0_v, d1_v, g0x, g1x, osub, sem0, sem1):
    wid = lax.axis_index("s") * NC + lax.axis_index("c")
    base = wid * TPW
    pltpu.sync_copy(d0_hbm.at[wid], d0_v)
    pltpu.sync_copy(d1_hbm.at[wid], d1_v)
    pltpu.async_copy(osr_hbm.at[d0_v.at[0]], g0x.at[0], sem0)
    pltpu.async_copy(osr_hbm.at[d1_v.at[0]], g1x.at[0], sem1)

    def sub(j, carry):
        s = j % 2
        g0 = g0x.at[s]
        g1 = g1x.at[s]
        pltpu.make_async_copy(osr_hbm.at[d0_v.at[j]], g0, sem0).wait()
        pltpu.make_async_copy(osr_hbm.at[d1_v.at[j]], g1, sem1).wait()
        jn = (j + 1) % NSUB
        sn = (j + 1) % 2
        pltpu.async_copy(osr_hbm.at[d0_v.at[jn]], g0x.at[sn], sem0)
        pltpu.async_copy(osr_hbm.at[d1_v.at[jn]], g1x.at[sn], sem1)

        def body(t, _):
            for q in range(PACKW // 16):
                cs = pl.ds(q * 16, 16)
                a = g0[t, cs]
                b = g1[t, cs]
                osub[t, cs] = (
                    lax.bitcast_convert_type(lax.shift_left(a, 16),
                                             jnp.float32)
                    + lax.bitcast_convert_type(lax.shift_left(b, 16),
                                               jnp.float32))
                osub[t, pl.ds(PACKW + q * 16, 16)] = (
                    lax.bitcast_convert_type(a & jnp.int32(-65536),
                                             jnp.float32)
                    + lax.bitcast_convert_type(b & jnp.int32(-65536),
                                               jnp.float32))
            return 0

        lax.fori_loop(0, SUB, body, 0)
        pltpu.sync_copy(osub, out_hbm.at[pl.ds(base + j * SUB, SUB)])
        return carry

    lax.fori_loop(0, NSUB, sub, 0)
    pltpu.make_async_copy(osr_hbm.at[d0_v.at[0]], g0x.at[0], sem0).wait()
    pltpu.make_async_copy(osr_hbm.at[d1_v.at[0]], g1x.at[0], sem1).wait()


def _combine(osr, d0, d1):
    mesh = plsc.VectorSubcoreMesh(core_axis_name="c", subcore_axis_name="s")
    f = pl.kernel(
        _comb_body,
        out_type=jax.ShapeDtypeStruct((N_TOK, D_MODEL), jnp.float32),
        mesh=mesh,
        scratch_types=[
            pltpu.VMEM((NSUB, SUB), jnp.int32),
            pltpu.VMEM((NSUB, SUB), jnp.int32),
            pltpu.VMEM((2, SUB, PACKW), jnp.int32),
            pltpu.VMEM((2, SUB, PACKW), jnp.int32),
            pltpu.VMEM((SUB, D_MODEL), jnp.float32),
            pltpu.SemaphoreType.DMA,
            pltpu.SemaphoreType.DMA,
        ],
        compiler_params=pltpu.CompilerParams(needs_layout_passes=False),
    )
    return f(osr, d0, d1)


def kernel(x, Wg, W1, b1, W2, b2):
    e1o, e2o, r0o, r1o, wso, x16, poffo, teo = _gating(x, Wg)
    e1 = e1o.reshape(N_TOK)
    e2 = e2o.reshape(N_TOK)
    r0 = r0o.reshape(N_TOK)
    r1 = r1o.reshape(N_TOK)
    ws = wso.reshape(N_TOK)
    poff = poffo.reshape(128)
    te = teo.reshape(512)
    xs, wss, d0, d1 = _dispatch(x16, e1, e2, r0, r1, poff, ws)
    osr = _grouped_ffn(te, xs, W1, b1, W2, b2, wss)
    return _combine(osr, d0, d1)
